# Initial kernel scaffold; baseline (speedup 1.0000x reference)
import functools

import jax
import jax.numpy as jnp
from jax import lax
from jax.experimental import pallas as pl
from jax.experimental.pallas import tpu as pltpu
from jax.experimental.pallas import tpu_sc as plsc

N = 10000
E = 320000
D = 128

NC = 2
NS = 16
NW = NC * NS
EPW = E // NW
BATCH = 80
NBATCH = EPW // BATCH
DEG_PAD = 10240
DEG_PW = DEG_PAD // NS
ROWS_PW = N // NS

ROW_BLK = 1000
GRID = N // ROW_BLK

_mesh = plsc.VectorSubcoreMesh(core_axis_name="c", subcore_axis_name="s")


@functools.partial(
    pl.kernel,
    mesh=_mesh,
    out_type=jax.ShapeDtypeStruct((NC, DEG_PAD), jnp.float32),
    scratch_types=[
        pltpu.VMEM_SHARED((DEG_PAD,), jnp.float32),
        pltpu.VMEM((BATCH,), jnp.int32),
        pltpu.VMEM((BATCH,), jnp.float32),
    ],
)
def _deg_sc(dst_hbm, ones_hbm, out_hbm, deg_sp, dst_v, ones_v):
    c = lax.axis_index("c")
    s = lax.axis_index("s")
    wid = s * NC + c
    pltpu.sync_copy(ones_hbm.at[pl.ds(s * DEG_PW, DEG_PW)],
                    deg_sp.at[pl.ds(s * DEG_PW, DEG_PW)])
    pltpu.sync_copy(ones_hbm.at[pl.ds(0, BATCH)], ones_v)
    plsc.subcore_barrier()

    def body(j, carry):
        base = wid * EPW + j * BATCH
        pltpu.sync_copy(dst_hbm.at[pl.ds(base, BATCH)], dst_v)
        pltpu.sync_copy(ones_v, deg_sp.at[dst_v], add=True)
        return carry

    lax.fori_loop(0, NBATCH, body, 0)
    plsc.subcore_barrier()
    pltpu.sync_copy(deg_sp.at[pl.ds(s * DEG_PW, DEG_PW)],
                    out_hbm.at[c, pl.ds(s * DEG_PW, DEG_PW)])


@functools.partial(
    pl.kernel,
    mesh=_mesh,
    out_type=jax.ShapeDtypeStruct((NC, N, D), jnp.float32),
    scratch_types=[
        pltpu.VMEM_SHARED((N, D), jnp.float32),
        pltpu.VMEM((BATCH,), jnp.int32),
        pltpu.VMEM((BATCH,), jnp.int32),
        pltpu.VMEM((BATCH, D), jnp.float32),
        pltpu.SemaphoreType.DMA,
    ],
)
def _agg_sc(h_hbm, src_hbm, dst_hbm, out_hbm, acc_sp, src_v, dst_v, rows_v, sem):
    c = lax.axis_index("c")
    s = lax.axis_index("s")
    wid = s * NC + c
    pltpu.sync_copy(h_hbm.at[pl.ds(s * ROWS_PW, ROWS_PW)],
                    acc_sp.at[pl.ds(s * ROWS_PW, ROWS_PW)])
    plsc.subcore_barrier()

    def body(j, carry):
        base = wid * EPW + j * BATCH
        pltpu.sync_copy(src_hbm.at[pl.ds(base, BATCH)], src_v)
        pltpu.sync_copy(dst_hbm.at[pl.ds(base, BATCH)], dst_v)
        pltpu.async_copy(h_hbm.at[src_v], rows_v, sem).wait()
        pltpu.sync_copy(rows_v, acc_sp.at[dst_v], add=True)
        return carry

    lax.fori_loop(0, NBATCH, body, 0)
    plsc.subcore_barrier()
    pltpu.sync_copy(acc_sp.at[pl.ds(s * ROWS_PW, ROWS_PW)],
                    out_hbm.at[c, pl.ds(s * ROWS_PW, ROWS_PW)])


def _dis(d0_ref, d1_ref):
    deg = d0_ref[0, 0, :] + d1_ref[0, 0, :] - 1.0
    return lax.rsqrt(deg)


def _lin1_body(x_ref, d0_ref, d1_ref, w_ref, h_ref):
    dis = _dis(d0_ref, d1_ref)
    h = jnp.dot(x_ref[...], w_ref[...], preferred_element_type=jnp.float32,
                precision=lax.Precision.HIGHEST)
    h_ref[...] = h * dis[:, None]


def _lin2_body(a0_ref, a1_ref, h2_ref, d0_ref, d1_ref, b_ref, w_ref, out_ref):
    dis = _dis(d0_ref, d1_ref)
    tot = a0_ref[...] + a1_ref[...] - h2_ref[...]
    u = jnp.maximum(tot * dis[:, None] + b_ref[0, 0, :][None, :], 0.0)
    h = jnp.dot(u, w_ref[...], preferred_element_type=jnp.float32,
                precision=lax.Precision.HIGHEST)
    out_ref[...] = h * dis[:, None]


def _out_body(a0_ref, a1_ref, h3_ref, d0_ref, d1_ref, b_ref, out_ref):
    dis = _dis(d0_ref, d1_ref)
    tot = a0_ref[...] + a1_ref[...] - h3_ref[...]
    out_ref[...] = tot * dis[:, None] + b_ref[0, 0, :][None, :]


_row_spec = pl.BlockSpec((ROW_BLK, D), lambda i: (i, 0))
_deg_spec = pl.BlockSpec((1, 1, ROW_BLK), lambda i: (i, 0, 0))
_w_spec = pl.BlockSpec((D, D), lambda i: (0, 0))
_b_spec = pl.BlockSpec((1, 1, D), lambda i: (0, 0, 0))

_lin1 = pl.pallas_call(
    _lin1_body,
    grid=(GRID,),
    in_specs=[_row_spec, _deg_spec, _deg_spec, _w_spec],
    out_specs=_row_spec,
    out_shape=jax.ShapeDtypeStruct((N, D), jnp.float32),
)

_lin2 = pl.pallas_call(
    _lin2_body,
    grid=(GRID,),
    in_specs=[_row_spec, _row_spec, _row_spec, _deg_spec, _deg_spec, _b_spec, _w_spec],
    out_specs=_row_spec,
    out_shape=jax.ShapeDtypeStruct((N, D), jnp.float32),
)

_outk = pl.pallas_call(
    _out_body,
    grid=(GRID,),
    in_specs=[_row_spec, _row_spec, _row_spec, _deg_spec, _deg_spec, _b_spec],
    out_specs=_row_spec,
    out_shape=jax.ShapeDtypeStruct((N, D), jnp.float32),
)


@jax.jit
def kernel(x, edge_index, W1, b1, W2, b2):
    src = edge_index[0]
    dst = edge_index[1]
    ones = jnp.ones((DEG_PAD,), jnp.float32)

    deg_parts = _deg_sc(dst, ones)
    d0 = deg_parts[0, :N].reshape(GRID, 1, ROW_BLK)
    d1 = deg_parts[1, :N].reshape(GRID, 1, ROW_BLK)
    b1r = b1.reshape(1, 1, D)
    b2r = b2.reshape(1, 1, D)

    h2 = _lin1(x, d0, d1, W1)
    acc = _agg_sc(h2, src, dst)
    h3 = _lin2(acc[0], acc[1], h2, d0, d1, b1r, W2)
    acc2 = _agg_sc(h3, src, dst)
    return _outk(acc2[0], acc2[1], h3, d0, d1, b2r)


# speedup vs baseline: 12.8564x; 12.8564x over previous
import functools

import jax
import jax.numpy as jnp
from jax import lax
from jax.experimental import pallas as pl
from jax.experimental.pallas import tpu as pltpu
from jax.experimental.pallas import tpu_sc as plsc

N = 10000
NP = 10240
E = 320000
D = 128

NC = 2
NS = 16
NW = NC * NS
EPW = E // NW
BATCH = 80
NBATCH = EPW // BATCH
DEG_PW = NP // NS
ROWS_PW = NP // NS

ROW_BLK = 1024
GRID = NP // ROW_BLK


@functools.cache
def _get_deg_sc():
    mesh = plsc.VectorSubcoreMesh(core_axis_name="c", subcore_axis_name="s")
    return pl.kernel(
        _deg_sc_body,
        mesh=mesh,
        out_type=jax.ShapeDtypeStruct((NC, NP), jnp.float32),
        scratch_types=[
            pltpu.VMEM_SHARED((NP,), jnp.float32),
            pltpu.VMEM((BATCH,), jnp.int32),
            pltpu.VMEM((BATCH,), jnp.float32),
        ],
    )


def _deg_sc_body(dst_hbm, ones_hbm, out_hbm, deg_sp, dst_v, ones_v):
    c = lax.axis_index("c")
    s = lax.axis_index("s")
    wid = s * NC + c
    pltpu.sync_copy(ones_hbm.at[pl.ds(s * DEG_PW, DEG_PW)],
                    deg_sp.at[pl.ds(s * DEG_PW, DEG_PW)])
    pltpu.sync_copy(ones_hbm.at[pl.ds(0, BATCH)], ones_v)
    plsc.subcore_barrier()

    def body(j, carry):
        base = wid * EPW + j * BATCH
        pltpu.sync_copy(dst_hbm.at[pl.ds(base, BATCH)], dst_v)
        pltpu.sync_copy(ones_v, deg_sp.at[dst_v], add=True)
        return carry

    lax.fori_loop(0, NBATCH, body, 0)
    plsc.subcore_barrier()
    pltpu.sync_copy(deg_sp.at[pl.ds(s * DEG_PW, DEG_PW)],
                    out_hbm.at[c, pl.ds(s * DEG_PW, DEG_PW)])


@functools.cache
def _get_agg_sc():
    mesh = plsc.VectorSubcoreMesh(core_axis_name="c", subcore_axis_name="s")
    return pl.kernel(
        _agg_sc_body,
        mesh=mesh,
        out_type=jax.ShapeDtypeStruct((NC, NP, D), jnp.float32),
        scratch_types=[
            pltpu.VMEM_SHARED((NP, D), jnp.float32),
            pltpu.VMEM((BATCH,), jnp.int32),
            pltpu.VMEM((BATCH,), jnp.int32),
            pltpu.VMEM((BATCH, D), jnp.float32),
            pltpu.SemaphoreType.DMA,
        ],
    )


def _agg_sc_body(h_hbm, src_hbm, dst_hbm, out_hbm, acc_sp, src_v, dst_v, rows_v, sem):
    c = lax.axis_index("c")
    s = lax.axis_index("s")
    wid = s * NC + c
    pltpu.sync_copy(h_hbm.at[pl.ds(s * ROWS_PW, ROWS_PW)],
                    acc_sp.at[pl.ds(s * ROWS_PW, ROWS_PW)])
    plsc.subcore_barrier()

    def body(j, carry):
        base = wid * EPW + j * BATCH
        pltpu.sync_copy(src_hbm.at[pl.ds(base, BATCH)], src_v)
        pltpu.sync_copy(dst_hbm.at[pl.ds(base, BATCH)], dst_v)
        pltpu.async_copy(h_hbm.at[src_v], rows_v, sem).wait()
        pltpu.sync_copy(rows_v, acc_sp.at[dst_v], add=True)
        return carry

    lax.fori_loop(0, NBATCH, body, 0)
    plsc.subcore_barrier()
    pltpu.sync_copy(acc_sp.at[pl.ds(s * ROWS_PW, ROWS_PW)],
                    out_hbm.at[c, pl.ds(s * ROWS_PW, ROWS_PW)])


def _dis(d0_ref, d1_ref):
    deg = d0_ref[0, 0, :] + d1_ref[0, 0, :] - 1.0
    return lax.rsqrt(deg)


def _lin1_body(x_ref, d0_ref, d1_ref, w_ref, h_ref):
    dis = _dis(d0_ref, d1_ref)
    h = jnp.dot(x_ref[...], w_ref[...], preferred_element_type=jnp.float32,
                precision=lax.Precision.HIGHEST)
    h_ref[...] = h * dis[:, None]


def _lin2_body(a0_ref, a1_ref, h2_ref, d0_ref, d1_ref, b_ref, w_ref, out_ref):
    dis = _dis(d0_ref, d1_ref)
    tot = a0_ref[...] + a1_ref[...] - h2_ref[...]
    u = jnp.maximum(tot * dis[:, None] + b_ref[0, 0, :][None, :], 0.0)
    h = jnp.dot(u, w_ref[...], preferred_element_type=jnp.float32,
                precision=lax.Precision.HIGHEST)
    out_ref[...] = h * dis[:, None]


def _out_body(a0_ref, a1_ref, h3_ref, d0_ref, d1_ref, b_ref, out_ref):
    dis = _dis(d0_ref, d1_ref)
    tot = a0_ref[...] + a1_ref[...] - h3_ref[...]
    out_ref[...] = tot * dis[:, None] + b_ref[0, 0, :][None, :]


_row_spec = pl.BlockSpec((ROW_BLK, D), lambda i: (i, 0))
_deg_spec = pl.BlockSpec((1, 1, ROW_BLK), lambda i: (i, 0, 0))
_w_spec = pl.BlockSpec((D, D), lambda i: (0, 0))
_b_spec = pl.BlockSpec((1, 1, D), lambda i: (0, 0, 0))

_lin1 = pl.pallas_call(
    _lin1_body,
    grid=(GRID,),
    in_specs=[_row_spec, _deg_spec, _deg_spec, _w_spec],
    out_specs=_row_spec,
    out_shape=jax.ShapeDtypeStruct((NP, D), jnp.float32),
)

_lin2 = pl.pallas_call(
    _lin2_body,
    grid=(GRID,),
    in_specs=[_row_spec, _row_spec, _row_spec, _deg_spec, _deg_spec, _b_spec, _w_spec],
    out_specs=_row_spec,
    out_shape=jax.ShapeDtypeStruct((NP, D), jnp.float32),
)

_outk = pl.pallas_call(
    _out_body,
    grid=(GRID,),
    in_specs=[_row_spec, _row_spec, _row_spec, _deg_spec, _deg_spec, _b_spec],
    out_specs=_row_spec,
    out_shape=jax.ShapeDtypeStruct((NP, D), jnp.float32),
)


@jax.jit
def kernel(x, edge_index, W1, b1, W2, b2):
    src = edge_index[0]
    dst = edge_index[1]
    ones = jnp.ones((NP,), jnp.float32)
    x_p = jnp.zeros((NP, D), jnp.float32).at[:N].set(x)

    deg_parts = _get_deg_sc()(dst, ones)
    d0 = deg_parts[0].reshape(GRID, 1, ROW_BLK)
    d1 = deg_parts[1].reshape(GRID, 1, ROW_BLK)
    b1r = b1.reshape(1, 1, D)
    b2r = b2.reshape(1, 1, D)

    agg = _get_agg_sc()
    h2 = _lin1(x_p, d0, d1, W1)
    acc = agg(h2, src, dst)
    h3 = _lin2(acc[0], acc[1], h2, d0, d1, b1r, W2)
    acc2 = agg(h3, src, dst)
    return _outk(acc2[0], acc2[1], h3, d0, d1, b2r)[:N]


# speedup vs baseline: 28.8800x; 2.2464x over previous
import functools

import jax
import jax.numpy as jnp
from jax import lax
from jax.experimental import pallas as pl
from jax.experimental.pallas import tpu as pltpu
from jax.experimental.pallas import tpu_sc as plsc

N = 10000
NP = 10240
E = 320000
D = 128

NC = 2
NS = 16
NW = NC * NS
EPW = E // NW
BATCH = 80
NBATCH = EPW // BATCH
DEG_PW = NP // NS
ROWS_PW = NP // NS

ROW_BLK = 1024
GRID = NP // ROW_BLK


@functools.cache
def _get_deg_sc():
    mesh = plsc.VectorSubcoreMesh(core_axis_name="c", subcore_axis_name="s")
    return pl.kernel(
        _deg_sc_body,
        mesh=mesh,
        out_type=jax.ShapeDtypeStruct((NC, NP), jnp.float32),
        scratch_types=[
            pltpu.VMEM_SHARED((NP,), jnp.float32),
            pltpu.VMEM((NBATCH, BATCH), jnp.int32),
            pltpu.VMEM((BATCH,), jnp.float32),
        ],
    )


def _deg_sc_body(dst_hbm, ones_hbm, out_hbm, deg_sp, dst_v, ones_v):
    c = lax.axis_index("c")
    s = lax.axis_index("s")
    wid = s * NC + c
    pltpu.sync_copy(ones_hbm.at[pl.ds(s * DEG_PW, DEG_PW)],
                    deg_sp.at[pl.ds(s * DEG_PW, DEG_PW)])
    pltpu.sync_copy(ones_hbm.at[pl.ds(0, BATCH)], ones_v)
    pltpu.sync_copy(dst_hbm.at[wid], dst_v)
    plsc.subcore_barrier()

    def body(j, carry):
        pltpu.sync_copy(ones_v, deg_sp.at[dst_v.at[j]], add=True)
        return carry

    lax.fori_loop(0, NBATCH, body, 0)
    plsc.subcore_barrier()
    pltpu.sync_copy(deg_sp.at[pl.ds(s * DEG_PW, DEG_PW)],
                    out_hbm.at[c, pl.ds(s * DEG_PW, DEG_PW)])


@functools.cache
def _get_agg_sc():
    mesh = plsc.VectorSubcoreMesh(core_axis_name="c", subcore_axis_name="s")
    return pl.kernel(
        _agg_sc_body,
        mesh=mesh,
        out_type=jax.ShapeDtypeStruct((NC, NP, D), jnp.float32),
        scratch_types=[
            pltpu.VMEM_SHARED((NP, D), jnp.float32),
            pltpu.VMEM((EPW,), jnp.int32),
            pltpu.VMEM((NBATCH, BATCH), jnp.int32),
            pltpu.VMEM((2, BATCH, D), jnp.float32),
            pltpu.SemaphoreType.DMA((2,)),
        ],
    )


def _agg_sc_body(h_hbm, src_hbm, dst_hbm, out_hbm, acc_sp, src_v, dst_v, rows_v, sem):
    c = lax.axis_index("c")
    s = lax.axis_index("s")
    wid = s * NC + c
    pltpu.sync_copy(h_hbm.at[pl.ds(s * ROWS_PW, ROWS_PW)],
                    acc_sp.at[pl.ds(s * ROWS_PW, ROWS_PW)])
    pltpu.sync_copy(src_hbm.at[wid], src_v)
    pltpu.sync_copy(dst_hbm.at[wid], dst_v)
    plsc.subcore_barrier()

    def gather_start(j, p):
        pltpu.async_copy(h_hbm.at[src_v.at[pl.ds(j * BATCH, BATCH)]],
                         rows_v.at[p], sem.at[p])

    def gather_wait(p):
        pltpu.make_async_copy(h_hbm.at[pl.ds(0, BATCH)],
                              rows_v.at[p], sem.at[p]).wait()

    gather_start(0, 0)
    gather_start(1, 1)

    def body(j, carry):
        p = lax.rem(j, 2)
        gather_wait(p)
        pltpu.sync_copy(rows_v.at[p], acc_sp.at[dst_v.at[j]], add=True)

        @pl.when(j + 2 < NBATCH)
        def _():
            gather_start(j + 2, p)

        return carry

    lax.fori_loop(0, NBATCH, body, 0)
    plsc.subcore_barrier()
    pltpu.sync_copy(acc_sp.at[pl.ds(s * ROWS_PW, ROWS_PW)],
                    out_hbm.at[c, pl.ds(s * ROWS_PW, ROWS_PW)])


def _dis(d0_ref, d1_ref):
    deg = d0_ref[0, 0, :] + d1_ref[0, 0, :] - 1.0
    return lax.rsqrt(deg)


def _lin1_body(x_ref, d0_ref, d1_ref, w_ref, h_ref):
    dis = _dis(d0_ref, d1_ref)
    h = jnp.dot(x_ref[...], w_ref[...], preferred_element_type=jnp.float32,
                precision=lax.Precision.HIGHEST)
    h_ref[...] = h * dis[:, None]


def _lin2_body(a0_ref, a1_ref, h2_ref, d0_ref, d1_ref, b_ref, w_ref, out_ref):
    dis = _dis(d0_ref, d1_ref)
    tot = a0_ref[...] + a1_ref[...] - h2_ref[...]
    u = jnp.maximum(tot * dis[:, None] + b_ref[0, 0, :][None, :], 0.0)
    h = jnp.dot(u, w_ref[...], preferred_element_type=jnp.float32,
                precision=lax.Precision.HIGHEST)
    out_ref[...] = h * dis[:, None]


def _out_body(a0_ref, a1_ref, h3_ref, d0_ref, d1_ref, b_ref, out_ref):
    dis = _dis(d0_ref, d1_ref)
    tot = a0_ref[...] + a1_ref[...] - h3_ref[...]
    out_ref[...] = tot * dis[:, None] + b_ref[0, 0, :][None, :]


_row_spec = pl.BlockSpec((ROW_BLK, D), lambda i: (i, 0))
_deg_spec = pl.BlockSpec((1, 1, ROW_BLK), lambda i: (i, 0, 0))
_w_spec = pl.BlockSpec((D, D), lambda i: (0, 0))
_b_spec = pl.BlockSpec((1, 1, D), lambda i: (0, 0, 0))

_lin1 = pl.pallas_call(
    _lin1_body,
    grid=(GRID,),
    in_specs=[_row_spec, _deg_spec, _deg_spec, _w_spec],
    out_specs=_row_spec,
    out_shape=jax.ShapeDtypeStruct((NP, D), jnp.float32),
)

_lin2 = pl.pallas_call(
    _lin2_body,
    grid=(GRID,),
    in_specs=[_row_spec, _row_spec, _row_spec, _deg_spec, _deg_spec, _b_spec, _w_spec],
    out_specs=_row_spec,
    out_shape=jax.ShapeDtypeStruct((NP, D), jnp.float32),
)

_outk = pl.pallas_call(
    _out_body,
    grid=(GRID,),
    in_specs=[_row_spec, _row_spec, _row_spec, _deg_spec, _deg_spec, _b_spec],
    out_specs=_row_spec,
    out_shape=jax.ShapeDtypeStruct((NP, D), jnp.float32),
)


@jax.jit
def kernel(x, edge_index, W1, b1, W2, b2):
    src = edge_index[0].reshape(NW, EPW)
    dst = edge_index[1].reshape(NW, NBATCH, BATCH)
    ones = jnp.ones((NP,), jnp.float32)
    x_p = jnp.zeros((NP, D), jnp.float32).at[:N].set(x)

    deg_parts = _get_deg_sc()(dst, ones)
    d0 = deg_parts[0].reshape(GRID, 1, ROW_BLK)
    d1 = deg_parts[1].reshape(GRID, 1, ROW_BLK)
    b1r = b1.reshape(1, 1, D)
    b2r = b2.reshape(1, 1, D)

    agg = _get_agg_sc()
    h2 = _lin1(x_p, d0, d1, W1)
    acc = agg(h2, src, dst)
    h3 = _lin2(acc[0], acc[1], h2, d0, d1, b1r, W2)
    acc2 = agg(h3, src, dst)
    return _outk(acc2[0], acc2[1], h3, d0, d1, b2r)[:N]


# speedup vs baseline: 29.6274x; 1.0259x over previous
import functools

import jax
import jax.numpy as jnp
from jax import lax
from jax.experimental import pallas as pl
from jax.experimental.pallas import tpu as pltpu
from jax.experimental.pallas import tpu_sc as plsc

N = 10000
NP = 10240
E = 320000
D = 128

NC = 2
NS = 16
NW = NC * NS
EPW = E // NW
BATCH = 80
NBATCH = EPW // BATCH
DEG_PW = NP // NS
ROWS_PW = NP // NS

NBUF = 3
ROW_BLK = 1024
GRID = NP // ROW_BLK


@functools.cache
def _get_deg_sc():
    mesh = plsc.VectorSubcoreMesh(core_axis_name="c", subcore_axis_name="s")
    return pl.kernel(
        _deg_sc_body,
        mesh=mesh,
        out_type=jax.ShapeDtypeStruct((NC, NP), jnp.float32),
        scratch_types=[
            pltpu.VMEM_SHARED((NP,), jnp.float32),
            pltpu.VMEM((NBATCH, BATCH), jnp.int32),
            pltpu.VMEM((BATCH,), jnp.float32),
        ],
    )


def _deg_sc_body(dst_hbm, ones_hbm, out_hbm, deg_sp, dst_v, ones_v):
    c = lax.axis_index("c")
    s = lax.axis_index("s")
    wid = s * NC + c
    pltpu.sync_copy(ones_hbm.at[pl.ds(s * DEG_PW, DEG_PW)],
                    deg_sp.at[pl.ds(s * DEG_PW, DEG_PW)])
    pltpu.sync_copy(ones_hbm.at[pl.ds(0, BATCH)], ones_v)
    pltpu.sync_copy(dst_hbm.at[wid], dst_v)
    plsc.subcore_barrier()

    def body(j, carry):
        pltpu.sync_copy(ones_v, deg_sp.at[dst_v.at[j]], add=True)
        return carry

    lax.fori_loop(0, NBATCH, body, 0)
    plsc.subcore_barrier()
    pltpu.sync_copy(deg_sp.at[pl.ds(s * DEG_PW, DEG_PW)],
                    out_hbm.at[c, pl.ds(s * DEG_PW, DEG_PW)])


@functools.cache
def _get_agg_sc():
    mesh = plsc.VectorSubcoreMesh(core_axis_name="c", subcore_axis_name="s")
    return pl.kernel(
        _agg_sc_body,
        mesh=mesh,
        out_type=jax.ShapeDtypeStruct((NC, NP, D), jnp.float32),
        scratch_types=[
            pltpu.VMEM_SHARED((NP, D), jnp.float32),
            pltpu.VMEM((NBUF * BATCH,), jnp.int32),
            pltpu.VMEM((NBATCH, BATCH), jnp.int32),
            pltpu.VMEM((NBUF, BATCH, D), jnp.float32),
            pltpu.SemaphoreType.DMA((NBUF,)),
            pltpu.SemaphoreType.DMA((NBUF,)),
            pltpu.SemaphoreType.DMA((NBUF,)),
        ],
    )


def _agg_sc_body(h_hbm, src_hbm, dst_hbm, out_hbm, acc_sp, src_v, dst_v, rows_v,
                 gsem, ssem, isem):
    c = lax.axis_index("c")
    s = lax.axis_index("s")
    wid = s * NC + c
    pltpu.sync_copy(h_hbm.at[pl.ds(s * ROWS_PW, ROWS_PW)],
                    acc_sp.at[pl.ds(s * ROWS_PW, ROWS_PW)])
    pltpu.sync_copy(dst_hbm.at[wid], dst_v)

    def idx_start(k, p):
        pltpu.async_copy(src_hbm.at[pl.ds((wid * NBATCH + k) * BATCH, BATCH)],
                         src_v.at[pl.ds(p * BATCH, BATCH)], isem.at[p])

    def idx_wait(p):
        pltpu.make_async_copy(src_hbm.at[pl.ds(0, BATCH)],
                              src_v.at[pl.ds(p * BATCH, BATCH)],
                              isem.at[p]).wait()

    def gather_start(j, p):
        pltpu.async_copy(h_hbm.at[src_v.at[pl.ds(p * BATCH, BATCH)]],
                         rows_v.at[p], gsem.at[p])

    def gather_wait(p):
        pltpu.make_async_copy(h_hbm.at[pl.ds(0, BATCH)],
                              rows_v.at[p], gsem.at[p]).wait()

    def scatter_drain(q):
        pltpu.make_async_copy(h_hbm.at[pl.ds(0, BATCH)],
                              acc_sp.at[pl.ds(0, BATCH)], ssem.at[q]).wait()

    for p in range(NBUF):
        idx_start(p, p)
    plsc.subcore_barrier()
    for p in range(NBUF - 1):
        idx_wait(p)
        gather_start(p, p)

    def body(j, carry):
        p = lax.rem(j, NBUF)
        gather_wait(p)
        pltpu.sync_copy(rows_v.at[p], acc_sp.at[dst_v.at[j]], add=True)

        @pl.when(j + NBUF - 1 < NBATCH)
        def _():
            q = lax.rem(j + NBUF - 1, NBUF)
            idx_wait(q)
            gather_start(j + NBUF - 1, q)

        @pl.when(j + NBUF < NBATCH)
        def _():
            idx_start(j + NBUF, p)

        return carry

    lax.fori_loop(0, NBATCH, body, 0)
    plsc.subcore_barrier()
    pltpu.sync_copy(acc_sp.at[pl.ds(s * ROWS_PW, ROWS_PW)],
                    out_hbm.at[c, pl.ds(s * ROWS_PW, ROWS_PW)])


def _dis(d0_ref, d1_ref):
    deg = d0_ref[0, 0, :] + d1_ref[0, 0, :] - 1.0
    return lax.rsqrt(deg)


def _lin1_body(x_ref, d0_ref, d1_ref, w_ref, h_ref):
    dis = _dis(d0_ref, d1_ref)
    h = jnp.dot(x_ref[...], w_ref[...], preferred_element_type=jnp.float32,
                precision=lax.Precision.HIGHEST)
    h_ref[...] = h * dis[:, None]


def _lin2_body(a0_ref, a1_ref, h2_ref, d0_ref, d1_ref, b_ref, w_ref, out_ref):
    dis = _dis(d0_ref, d1_ref)
    tot = a0_ref[...] + a1_ref[...] - h2_ref[...]
    u = jnp.maximum(tot * dis[:, None] + b_ref[0, 0, :][None, :], 0.0)
    h = jnp.dot(u, w_ref[...], preferred_element_type=jnp.float32,
                precision=lax.Precision.HIGHEST)
    out_ref[...] = h * dis[:, None]


def _out_body(a0_ref, a1_ref, h3_ref, d0_ref, d1_ref, b_ref, out_ref):
    dis = _dis(d0_ref, d1_ref)
    tot = a0_ref[...] + a1_ref[...] - h3_ref[...]
    out_ref[...] = tot * dis[:, None] + b_ref[0, 0, :][None, :]


_row_spec = pl.BlockSpec((ROW_BLK, D), lambda i: (i, 0))
_deg_spec = pl.BlockSpec((1, 1, ROW_BLK), lambda i: (i, 0, 0))
_w_spec = pl.BlockSpec((D, D), lambda i: (0, 0))
_b_spec = pl.BlockSpec((1, 1, D), lambda i: (0, 0, 0))

_lin1 = pl.pallas_call(
    _lin1_body,
    grid=(GRID,),
    in_specs=[_row_spec, _deg_spec, _deg_spec, _w_spec],
    out_specs=_row_spec,
    out_shape=jax.ShapeDtypeStruct((NP, D), jnp.float32),
)

_lin2 = pl.pallas_call(
    _lin2_body,
    grid=(GRID,),
    in_specs=[_row_spec, _row_spec, _row_spec, _deg_spec, _deg_spec, _b_spec, _w_spec],
    out_specs=_row_spec,
    out_shape=jax.ShapeDtypeStruct((NP, D), jnp.float32),
)

_outk = pl.pallas_call(
    _out_body,
    grid=(GRID,),
    in_specs=[_row_spec, _row_spec, _row_spec, _deg_spec, _deg_spec, _b_spec],
    out_specs=_row_spec,
    out_shape=jax.ShapeDtypeStruct((N, D), jnp.float32),
)


@jax.jit
def kernel(x, edge_index, W1, b1, W2, b2):
    src = edge_index[0]
    dst = edge_index[1].reshape(NW, NBATCH, BATCH)
    ones = jnp.ones((NP,), jnp.float32)

    deg_parts = _get_deg_sc()(dst, ones)
    d0 = deg_parts[0].reshape(GRID, 1, ROW_BLK)
    d1 = deg_parts[1].reshape(GRID, 1, ROW_BLK)
    b1r = b1.reshape(1, 1, D)
    b2r = b2.reshape(1, 1, D)

    agg = _get_agg_sc()
    h2 = _lin1(x, d0, d1, W1)
    acc = agg(h2, src, dst)
    h3 = _lin2(acc[0], acc[1], h2, d0, d1, b1r, W2)
    acc2 = agg(h3, src, dst)
    return _outk(acc2[0], acc2[1], h3, d0, d1, b2r)


# speedup vs baseline: 31.6807x; 1.0693x over previous
import functools

import jax
import jax.numpy as jnp
from jax import lax
from jax.experimental import pallas as pl
from jax.experimental.pallas import tpu as pltpu
from jax.experimental.pallas import tpu_sc as plsc

N = 10000
NP = 10240
E = 320000
D = 128

NC = 2
NS = 16
NW = NC * NS
EPW = E // NW
BATCH = 128
NBATCH = EPW // BATCH
TAIL = EPW - NBATCH * BATCH
DEG_PW = NP // NS
ROWS_PW = NP // NS

NBUF = 2
IBUF = 4
ROW_BLK = 1024
GRID = NP // ROW_BLK


@functools.cache
def _get_deg_sc():
    mesh = plsc.VectorSubcoreMesh(core_axis_name="c", subcore_axis_name="s")
    return pl.kernel(
        _deg_sc_body,
        mesh=mesh,
        out_type=jax.ShapeDtypeStruct((NC, NP), jnp.float32),
        scratch_types=[
            pltpu.VMEM_SHARED((NP,), jnp.float32),
            pltpu.VMEM((NBATCH, BATCH), jnp.int32),
            pltpu.VMEM((TAIL,), jnp.int32),
            pltpu.VMEM((BATCH,), jnp.float32),
        ],
    )


def _deg_sc_body(dst_hbm, dstt_hbm, ones_hbm, out_hbm, deg_sp, dst_v, dstt_v,
                 ones_v):
    c = lax.axis_index("c")
    s = lax.axis_index("s")
    wid = s * NC + c
    pltpu.sync_copy(ones_hbm.at[pl.ds(s * DEG_PW, DEG_PW)],
                    deg_sp.at[pl.ds(s * DEG_PW, DEG_PW)])
    pltpu.sync_copy(ones_hbm.at[pl.ds(0, BATCH)], ones_v)
    pltpu.sync_copy(dst_hbm.at[wid], dst_v)
    pltpu.sync_copy(dstt_hbm.at[wid], dstt_v)
    plsc.subcore_barrier()

    def body(j, carry):
        pltpu.sync_copy(ones_v, deg_sp.at[dst_v.at[j]], add=True)
        return carry

    lax.fori_loop(0, NBATCH, body, 0)
    pltpu.sync_copy(ones_v.at[pl.ds(0, TAIL)], deg_sp.at[dstt_v], add=True)
    plsc.subcore_barrier()
    pltpu.sync_copy(deg_sp.at[pl.ds(s * DEG_PW, DEG_PW)],
                    out_hbm.at[c, pl.ds(s * DEG_PW, DEG_PW)])


@functools.cache
def _get_agg_sc():
    mesh = plsc.VectorSubcoreMesh(core_axis_name="c", subcore_axis_name="s")
    return pl.kernel(
        _agg_sc_body,
        mesh=mesh,
        out_type=jax.ShapeDtypeStruct((NC, NP, D), jnp.float32),
        scratch_types=[
            pltpu.VMEM_SHARED((NP, D), jnp.float32),
            pltpu.VMEM((IBUF * BATCH,), jnp.int32),
            pltpu.VMEM((TAIL,), jnp.int32),
            pltpu.VMEM((NBATCH, BATCH), jnp.int32),
            pltpu.VMEM((TAIL,), jnp.int32),
            pltpu.VMEM((NBUF, BATCH, D), jnp.float32),
            pltpu.VMEM((TAIL, D), jnp.float32),
            pltpu.SemaphoreType.DMA((NBUF,)),
            pltpu.SemaphoreType.DMA((IBUF,)),
        ],
    )


def _agg_sc_body(h_hbm, src_hbm, srct_hbm, dst_hbm, dstt_hbm, out_hbm, acc_sp,
                 src_v, srct_v, dst_v, dstt_v, rows_v, rowst_v, gsem, isem):
    c = lax.axis_index("c")
    s = lax.axis_index("s")
    wid = s * NC + c
    pltpu.sync_copy(h_hbm.at[pl.ds(s * ROWS_PW, ROWS_PW)],
                    acc_sp.at[pl.ds(s * ROWS_PW, ROWS_PW)])
    pltpu.sync_copy(dst_hbm.at[wid], dst_v)
    pltpu.sync_copy(dstt_hbm.at[wid], dstt_v)
    pltpu.sync_copy(srct_hbm.at[wid], srct_v)

    def idx_start(k, p):
        pltpu.async_copy(src_hbm.at[pl.ds(wid * EPW + k * BATCH, BATCH)],
                         src_v.at[pl.ds(p * BATCH, BATCH)], isem.at[p])

    def idx_wait(p):
        pltpu.make_async_copy(src_hbm.at[pl.ds(0, BATCH)],
                              src_v.at[pl.ds(p * BATCH, BATCH)],
                              isem.at[p]).wait()

    def gather_start(p, ip):
        pltpu.async_copy(h_hbm.at[src_v.at[pl.ds(ip * BATCH, BATCH)]],
                         rows_v.at[p], gsem.at[p])

    def gather_wait(p):
        pltpu.make_async_copy(h_hbm.at[pl.ds(0, BATCH)],
                              rows_v.at[p], gsem.at[p]).wait()

    for p in range(min(IBUF, NBATCH)):
        idx_start(p, p)
    plsc.subcore_barrier()
    for p in range(NBUF):
        idx_wait(p)
        gather_start(p, p)

    def body(j, carry):
        p = lax.rem(j, NBUF)
        gather_wait(p)
        pltpu.sync_copy(rows_v.at[p], acc_sp.at[dst_v.at[j]], add=True)

        @pl.when(j + NBUF < NBATCH)
        def _():
            ip = lax.rem(j + NBUF, IBUF)
            idx_wait(ip)
            gather_start(p, ip)

        @pl.when(j + IBUF < NBATCH)
        def _():
            idx_start(j + IBUF, lax.rem(j, IBUF))

        return carry

    lax.fori_loop(0, NBATCH, body, 0)
    pltpu.async_copy(h_hbm.at[srct_v], rowst_v, gsem.at[0]).wait()
    pltpu.sync_copy(rowst_v, acc_sp.at[dstt_v], add=True)
    plsc.subcore_barrier()
    pltpu.sync_copy(acc_sp.at[pl.ds(s * ROWS_PW, ROWS_PW)],
                    out_hbm.at[c, pl.ds(s * ROWS_PW, ROWS_PW)])


def _dis(d0_ref, d1_ref):
    deg = d0_ref[0, 0, :] + d1_ref[0, 0, :] - 1.0
    return lax.rsqrt(deg)


def _lin1_body(x_ref, d0_ref, d1_ref, w_ref, h_ref):
    dis = _dis(d0_ref, d1_ref)
    h = jnp.dot(x_ref[...], w_ref[...], preferred_element_type=jnp.float32,
                precision=lax.Precision.HIGHEST)
    h_ref[...] = h * dis[:, None]


def _lin2_body(a0_ref, a1_ref, h2_ref, d0_ref, d1_ref, b_ref, w_ref, out_ref):
    dis = _dis(d0_ref, d1_ref)
    tot = a0_ref[...] + a1_ref[...] - h2_ref[...]
    u = jnp.maximum(tot * dis[:, None] + b_ref[0, 0, :][None, :], 0.0)
    h = jnp.dot(u, w_ref[...], preferred_element_type=jnp.float32,
                precision=lax.Precision.HIGHEST)
    out_ref[...] = h * dis[:, None]


def _out_body(a0_ref, a1_ref, h3_ref, d0_ref, d1_ref, b_ref, out_ref):
    dis = _dis(d0_ref, d1_ref)
    tot = a0_ref[...] + a1_ref[...] - h3_ref[...]
    out_ref[...] = tot * dis[:, None] + b_ref[0, 0, :][None, :]


_row_spec = pl.BlockSpec((ROW_BLK, D), lambda i: (i, 0))
_deg_spec = pl.BlockSpec((1, 1, ROW_BLK), lambda i: (i, 0, 0))
_w_spec = pl.BlockSpec((D, D), lambda i: (0, 0))
_b_spec = pl.BlockSpec((1, 1, D), lambda i: (0, 0, 0))

_lin1 = pl.pallas_call(
    _lin1_body,
    grid=(GRID,),
    in_specs=[_row_spec, _deg_spec, _deg_spec, _w_spec],
    out_specs=_row_spec,
    out_shape=jax.ShapeDtypeStruct((NP, D), jnp.float32),
)

_lin2 = pl.pallas_call(
    _lin2_body,
    grid=(GRID,),
    in_specs=[_row_spec, _row_spec, _row_spec, _deg_spec, _deg_spec, _b_spec, _w_spec],
    out_specs=_row_spec,
    out_shape=jax.ShapeDtypeStruct((NP, D), jnp.float32),
)

_outk = pl.pallas_call(
    _out_body,
    grid=(GRID,),
    in_specs=[_row_spec, _row_spec, _row_spec, _deg_spec, _deg_spec, _b_spec],
    out_specs=_row_spec,
    out_shape=jax.ShapeDtypeStruct((N, D), jnp.float32),
)


@jax.jit
def kernel(x, edge_index, W1, b1, W2, b2):
    src = edge_index[0]
    d2 = edge_index[1].reshape(NW, EPW)
    dst_main = d2[:, :NBATCH * BATCH].reshape(NW, NBATCH, BATCH)
    dst_tail = d2[:, NBATCH * BATCH:]
    s2 = src.reshape(NW, EPW)
    src_tail = s2[:, NBATCH * BATCH:]
    ones = jnp.ones((NP,), jnp.float32)

    deg_parts = _get_deg_sc()(dst_main, dst_tail, ones)
    d0 = deg_parts[0].reshape(GRID, 1, ROW_BLK)
    d1 = deg_parts[1].reshape(GRID, 1, ROW_BLK)
    b1r = b1.reshape(1, 1, D)
    b2r = b2.reshape(1, 1, D)

    agg = _get_agg_sc()
    h2 = _lin1(x, d0, d1, W1)
    acc = agg(h2, src, src_tail, dst_main, dst_tail)
    h3 = _lin2(acc[0], acc[1], h2, d0, d1, b1r, W2)
    acc2 = agg(h3, src, src_tail, dst_main, dst_tail)
    return _outk(acc2[0], acc2[1], h3, d0, d1, b2r)


# speedup vs baseline: 33.5826x; 1.0600x over previous
import functools

import jax
import jax.numpy as jnp
from jax import lax
from jax.experimental import pallas as pl
from jax.experimental.pallas import tpu as pltpu
from jax.experimental.pallas import tpu_sc as plsc

N = 10000
NP = 10240
E = 320000
D = 128

NC = 2
NS = 16
NW = NC * NS
EPW = E // NW
BATCH = 128
NBATCH = EPW // BATCH
TAIL = EPW - NBATCH * BATCH
DEG_PW = NP // NS
ROWS_PW = NP // NS

NBUF = 2
IBUF = 4
ROW_BLK = 1024
GRID = NP // ROW_BLK


@functools.cache
def _get_deg_sc():
    mesh = plsc.VectorSubcoreMesh(core_axis_name="c", subcore_axis_name="s")
    return pl.kernel(
        _deg_sc_body,
        mesh=mesh,
        out_type=jax.ShapeDtypeStruct((NC, NP), jnp.float32),
        scratch_types=[
            pltpu.VMEM_SHARED((NP,), jnp.float32),
            pltpu.VMEM((NBATCH, BATCH), jnp.int32),
            pltpu.VMEM((TAIL,), jnp.int32),
            pltpu.VMEM((BATCH,), jnp.float32),
        ],
    )


def _deg_sc_body(dst_hbm, dstt_hbm, ones_hbm, out_hbm, deg_sp, dst_v, dstt_v,
                 ones_v):
    c = lax.axis_index("c")
    s = lax.axis_index("s")
    wid = s * NC + c
    pltpu.sync_copy(ones_hbm.at[pl.ds(s * DEG_PW, DEG_PW)],
                    deg_sp.at[pl.ds(s * DEG_PW, DEG_PW)])
    pltpu.sync_copy(ones_hbm.at[pl.ds(0, BATCH)], ones_v)
    pltpu.sync_copy(dst_hbm.at[wid], dst_v)
    pltpu.sync_copy(dstt_hbm.at[wid], dstt_v)
    plsc.subcore_barrier()

    def body(j, carry):
        pltpu.sync_copy(ones_v, deg_sp.at[dst_v.at[j]], add=True)
        return carry

    lax.fori_loop(0, NBATCH, body, 0)
    pltpu.sync_copy(ones_v.at[pl.ds(0, TAIL)], deg_sp.at[dstt_v], add=True)
    plsc.subcore_barrier()
    pltpu.sync_copy(deg_sp.at[pl.ds(s * DEG_PW, DEG_PW)],
                    out_hbm.at[c, pl.ds(s * DEG_PW, DEG_PW)])


@functools.cache
def _get_agg_sc():
    mesh = plsc.VectorSubcoreMesh(core_axis_name="c", subcore_axis_name="s")
    return pl.kernel(
        _agg_sc_body,
        mesh=mesh,
        out_type=jax.ShapeDtypeStruct((NC, NP, D), jnp.float32),
        scratch_types=[
            pltpu.VMEM_SHARED((NP, D), jnp.float32),
            pltpu.VMEM((IBUF * BATCH,), jnp.int32),
            pltpu.VMEM((TAIL,), jnp.int32),
            pltpu.VMEM((NBATCH, BATCH), jnp.int32),
            pltpu.VMEM((TAIL,), jnp.int32),
            pltpu.VMEM((NBUF, BATCH, D), jnp.float32),
            pltpu.VMEM((TAIL, D), jnp.float32),
            pltpu.SemaphoreType.DMA((NBUF,)),
            pltpu.SemaphoreType.DMA((IBUF,)),
        ],
    )


def _agg_sc_body(h_hbm, src_hbm, srct_hbm, dst_hbm, dstt_hbm, out_hbm, acc_sp,
                 src_v, srct_v, dst_v, dstt_v, rows_v, rowst_v, gsem, isem):
    c = lax.axis_index("c")
    s = lax.axis_index("s")
    wid = s * NC + c
    pltpu.sync_copy(h_hbm.at[pl.ds(s * ROWS_PW, ROWS_PW)],
                    acc_sp.at[pl.ds(s * ROWS_PW, ROWS_PW)])
    pltpu.sync_copy(dst_hbm.at[wid], dst_v)
    pltpu.sync_copy(dstt_hbm.at[wid], dstt_v)
    pltpu.sync_copy(srct_hbm.at[wid], srct_v)

    def idx_start(k, p):
        pltpu.async_copy(src_hbm.at[pl.ds(wid * EPW + k * BATCH, BATCH)],
                         src_v.at[pl.ds(p * BATCH, BATCH)], isem.at[p])

    def idx_wait(p):
        pltpu.make_async_copy(src_hbm.at[pl.ds(0, BATCH)],
                              src_v.at[pl.ds(p * BATCH, BATCH)],
                              isem.at[p]).wait()

    def gather_start(p, ip):
        pltpu.async_copy(h_hbm.at[src_v.at[pl.ds(ip * BATCH, BATCH)]],
                         rows_v.at[p], gsem.at[p])

    def gather_wait(p):
        pltpu.make_async_copy(h_hbm.at[pl.ds(0, BATCH)],
                              rows_v.at[p], gsem.at[p]).wait()

    for p in range(min(IBUF, NBATCH)):
        idx_start(p, p)
    plsc.subcore_barrier()
    for p in range(NBUF):
        idx_wait(p)
        gather_start(p, p)

    def body(j, carry):
        p = lax.rem(j, NBUF)
        gather_wait(p)
        pltpu.sync_copy(rows_v.at[p], acc_sp.at[dst_v.at[j]], add=True)

        @pl.when(j + NBUF < NBATCH)
        def _():
            ip = lax.rem(j + NBUF, IBUF)
            idx_wait(ip)
            gather_start(p, ip)

        @pl.when(j + IBUF < NBATCH)
        def _():
            idx_start(j + IBUF, lax.rem(j, IBUF))

        return carry

    lax.fori_loop(0, NBATCH, body, 0)
    pltpu.async_copy(h_hbm.at[srct_v], rowst_v, gsem.at[0]).wait()
    pltpu.sync_copy(rowst_v, acc_sp.at[dstt_v], add=True)
    plsc.subcore_barrier()
    pltpu.sync_copy(acc_sp.at[pl.ds(s * ROWS_PW, ROWS_PW)],
                    out_hbm.at[c, pl.ds(s * ROWS_PW, ROWS_PW)])


def _dis(d0_ref, d1_ref):
    deg = d0_ref[0, 0, 0, :] + d1_ref[0, 0, 0, :] - 1.0
    return lax.rsqrt(deg)


def _lin1_body(x_ref, d0_ref, d1_ref, w_ref, h_ref):
    dis = _dis(d0_ref, d1_ref)
    h = jnp.dot(x_ref[...], w_ref[...], preferred_element_type=jnp.float32)
    h_ref[...] = h * dis[:, None]


def _lin2_body(a0_ref, a1_ref, h2_ref, d0_ref, d1_ref, b_ref, w_ref, out_ref):
    dis = _dis(d0_ref, d1_ref)
    tot = a0_ref[0] + a1_ref[0] - h2_ref[...]
    u = jnp.maximum(tot * dis[:, None] + b_ref[0, 0, :][None, :], 0.0)
    h = jnp.dot(u, w_ref[...], preferred_element_type=jnp.float32)
    out_ref[...] = h * dis[:, None]


def _out_body(a0_ref, a1_ref, h3_ref, d0_ref, d1_ref, b_ref, out_ref):
    dis = _dis(d0_ref, d1_ref)
    tot = a0_ref[0] + a1_ref[0] - h3_ref[...]
    out_ref[...] = tot * dis[:, None] + b_ref[0, 0, :][None, :]


_row_spec = pl.BlockSpec((ROW_BLK, D), lambda i: (i, 0))
_acc0_spec = pl.BlockSpec((1, ROW_BLK, D), lambda i: (0, i, 0))
_acc1_spec = pl.BlockSpec((1, ROW_BLK, D), lambda i: (1, i, 0))
_d0_spec = pl.BlockSpec((1, 1, 1, ROW_BLK), lambda i: (0, i, 0, 0))
_d1_spec = pl.BlockSpec((1, 1, 1, ROW_BLK), lambda i: (1, i, 0, 0))
_w_spec = pl.BlockSpec((D, D), lambda i: (0, 0))
_b_spec = pl.BlockSpec((1, 1, D), lambda i: (0, 0, 0))

_lin1 = pl.pallas_call(
    _lin1_body,
    grid=(GRID,),
    in_specs=[_row_spec, _d0_spec, _d1_spec, _w_spec],
    out_specs=_row_spec,
    out_shape=jax.ShapeDtypeStruct((NP, D), jnp.float32),
)

_lin2 = pl.pallas_call(
    _lin2_body,
    grid=(GRID,),
    in_specs=[_acc0_spec, _acc1_spec, _row_spec, _d0_spec, _d1_spec, _b_spec, _w_spec],
    out_specs=_row_spec,
    out_shape=jax.ShapeDtypeStruct((NP, D), jnp.float32),
)

_outk = pl.pallas_call(
    _out_body,
    grid=(GRID,),
    in_specs=[_acc0_spec, _acc1_spec, _row_spec, _d0_spec, _d1_spec, _b_spec],
    out_specs=_row_spec,
    out_shape=jax.ShapeDtypeStruct((N, D), jnp.float32),
)


@jax.jit
def kernel(x, edge_index, W1, b1, W2, b2):
    src = edge_index[0]
    d2 = edge_index[1].reshape(NW, EPW)
    dst_main = d2[:, :NBATCH * BATCH].reshape(NW, NBATCH, BATCH)
    dst_tail = d2[:, NBATCH * BATCH:]
    s2 = src.reshape(NW, EPW)
    src_tail = s2[:, NBATCH * BATCH:]
    ones = jnp.ones((NP,), jnp.float32)

    deg_parts = _get_deg_sc()(dst_main, dst_tail, ones)
    dp = deg_parts.reshape(2, GRID, 1, ROW_BLK)
    b1r = b1.reshape(1, 1, D)
    b2r = b2.reshape(1, 1, D)

    agg = _get_agg_sc()
    h2 = _lin1(x, dp, dp, W1)
    acc = agg(h2, src, src_tail, dst_main, dst_tail)
    h3 = _lin2(acc, acc, h2, dp, dp, b1r, W2)
    acc2 = agg(h3, src, src_tail, dst_main, dst_tail)
    return _outk(acc2, acc2, h3, dp, dp, b2r)
